# baseline (speedup 1.0000x reference)
import functools

import jax
import jax.numpy as jnp
from jax import lax
from jax.experimental import pallas as pl
from jax.experimental.pallas import tpu as pltpu, tpu_sc as plsc

_SQ3 = 0.5773502691896258

_NC, _NS, _L = 2, 16, 16
_NW = _NC * _NS
_NBUF = 5
_E = 10

_B_TC = 88000
_TB = 400


@functools.lru_cache(maxsize=None)
def _tp_sc_kernel(B, B_SC, C):
    XW = 4 * C
    OW = 8 * C
    b_per_w = B_SC // _NW
    E = _E
    assert B_SC % _NW == 0 and b_per_w % (_NBUF * E) == 0
    n_supers = b_per_w // E // _NBUF
    G = C // _L

    mesh = plsc.VectorSubcoreMesh(
        core_axis_name="c", subcore_axis_name="s",
        num_cores=_NC, num_subcores=_NS)

    @functools.partial(
        pl.kernel,
        out_type=jax.ShapeDtypeStruct((B * OW,), jnp.float32),
        mesh=mesh,
        scratch_types=(
            [pltpu.VMEM((E * XW,), jnp.float32) for _ in range(_NBUF)]
            + [pltpu.VMEM((E * XW,), jnp.float32) for _ in range(_NBUF)]
            + [pltpu.VMEM((E * OW,), jnp.float32) for _ in range(_NBUF)]
            + [pltpu.SemaphoreType.DMA for _ in range(3 * _NBUF)]
        ),
    )
    def k(x_hbm, y_hbm, o_hbm, *bufs):
        xvs = bufs[0:_NBUF]
        yvs = bufs[_NBUF:2 * _NBUF]
        ovs = bufs[2 * _NBUF:3 * _NBUF]
        sxs = bufs[3 * _NBUF:4 * _NBUF]
        sys_ = bufs[4 * _NBUF:5 * _NBUF]
        sos = bufs[5 * _NBUF:6 * _NBUF]

        wid = lax.axis_index("s") * _NC + lax.axis_index("c")
        base = wid * b_per_w

        def start_in(b, ci):
            e0 = (base + ci * E) * XW
            pltpu.make_async_copy(
                x_hbm.at[pl.ds(e0, E * XW)], xvs[b], sxs[b]).start()
            pltpu.make_async_copy(
                y_hbm.at[pl.ds(e0, E * XW)], yvs[b], sys_[b]).start()

        def wait_in(b):
            pltpu.make_async_copy(
                x_hbm.at[pl.ds(0, E * XW)], xvs[b], sxs[b]).wait()
            pltpu.make_async_copy(
                y_hbm.at[pl.ds(0, E * XW)], yvs[b], sys_[b]).wait()

        def start_out(b, ci):
            e0 = (base + ci * E) * OW
            pltpu.make_async_copy(
                ovs[b], o_hbm.at[pl.ds(e0, E * OW)], sos[b]).start()

        def wait_out(b):
            pltpu.make_async_copy(
                ovs[b], o_hbm.at[pl.ds(0, E * OW)], sos[b]).wait()

        def compute(b):
            xv, yv, ov = xvs[b], yvs[b], ovs[b]

            @plsc.parallel_loop(0, E, step=1, unroll=2)
            def edge_body(e):
                xb = e * XW
                ob = e * OW
                for g in range(G):
                    c0 = g * _L
                    x0 = xv[pl.ds(xb + 0 * C + c0, _L)]
                    x1 = xv[pl.ds(xb + 1 * C + c0, _L)]
                    x2 = xv[pl.ds(xb + 2 * C + c0, _L)]
                    x3 = xv[pl.ds(xb + 3 * C + c0, _L)]
                    y0 = yv[pl.ds(xb + 0 * C + c0, _L)]
                    y1 = yv[pl.ds(xb + 1 * C + c0, _L)]
                    y2 = yv[pl.ds(xb + 2 * C + c0, _L)]
                    y3 = yv[pl.ds(xb + 3 * C + c0, _L)]
                    ov[pl.ds(ob + 0 * C + c0, _L)] = x0 * y0
                    ov[pl.ds(ob + 1 * C + c0, _L)] = x0 * y1
                    ov[pl.ds(ob + 2 * C + c0, _L)] = x0 * y2
                    ov[pl.ds(ob + 3 * C + c0, _L)] = x0 * y3
                    ov[pl.ds(ob + 4 * C + c0, _L)] = x1 * y0
                    ov[pl.ds(ob + 5 * C + c0, _L)] = x2 * y0
                    ov[pl.ds(ob + 6 * C + c0, _L)] = x3 * y0
                    ov[pl.ds(ob + 7 * C + c0, _L)] = (
                        x1 * y1 + x2 * y2 + x3 * y3) * _SQ3

        for b in range(_NBUF):
            start_in(b, b)

        for b in range(_NBUF):
            wait_in(b)
            compute(b)
            start_out(b, b)
            start_in(b, b + _NBUF)

        def super_body(si, carry):
            for b in range(_NBUF):
                ci = si * _NBUF + b
                wait_in(b)
                wait_out(b)
                compute(b)
                start_out(b, ci)
                start_in(b, ci + _NBUF)
            return carry

        lax.fori_loop(1, n_supers - 1, super_body, 0)

        for b in range(_NBUF):
            wait_in(b)
            wait_out(b)
            compute(b)
            start_out(b, (n_supers - 1) * _NBUF + b)
        for b in range(_NBUF):
            wait_out(b)

    return k


def _tc_body(x_ref, y_ref, o_ref):
    C = 128
    x = x_ref[...]
    y = y_ref[...]
    x0 = x[:, 0 * C:1 * C]
    x1 = x[:, 1 * C:2 * C]
    x2 = x[:, 2 * C:3 * C]
    x3 = x[:, 3 * C:4 * C]
    y0 = y[:, 0 * C:1 * C]
    y1 = y[:, 1 * C:2 * C]
    y2 = y[:, 2 * C:3 * C]
    y3 = y[:, 3 * C:4 * C]
    o_ref[:, 0 * C:1 * C] = x0 * y0
    o_ref[:, 1 * C:2 * C] = x0 * y1
    o_ref[:, 2 * C:3 * C] = x0 * y2
    o_ref[:, 3 * C:4 * C] = x0 * y3
    o_ref[:, 4 * C:5 * C] = x1 * y0
    o_ref[:, 5 * C:6 * C] = x2 * y0
    o_ref[:, 6 * C:7 * C] = x3 * y0
    o_ref[:, 7 * C:8 * C] = (x1 * y1 + x2 * y2 + x3 * y3) * _SQ3


@functools.lru_cache(maxsize=None)
def _tp_tc_kernel(B, B_SC, B_TC, C):
    assert B_TC % _TB == 0 and B_SC % _TB == 0
    blk0 = B_SC // _TB
    grid = (B_TC // _TB,)
    return pl.pallas_call(
        _tc_body,
        grid=grid,
        in_specs=[
            pl.BlockSpec((_TB, 4 * C), lambda i: (blk0 + i, 0)),
            pl.BlockSpec((_TB, 4 * C), lambda i: (blk0 + i, 0)),
        ],
        out_specs=pl.BlockSpec((_TB, 8 * C), lambda i: (i, 0)),
        out_shape=jax.ShapeDtypeStruct((B_TC, 8 * C), jnp.float32),
    )


def kernel(x, y):
    B, _, C = x.shape
    B_TC = _B_TC
    B_SC = B - B_TC
    xf = x.reshape(B, 4 * C)
    yf = y.reshape(B, 4 * C)
    sc_full = _tp_sc_kernel(B, B_SC, C)(xf.reshape(-1), yf.reshape(-1))
    tc_part = _tp_tc_kernel(B, B_SC, B_TC, C)(xf, yf)
    out = lax.dynamic_update_slice(
        sc_full.reshape(B, 8 * C), tc_part, (B_SC, 0))
    return out.reshape(B, 8, C)


# speedup vs baseline: 2.7434x; 2.7434x over previous
import functools

import jax
import jax.numpy as jnp
from jax import lax
from jax.experimental import pallas as pl
from jax.experimental.pallas import tpu as pltpu, tpu_sc as plsc

_SQ3 = 0.5773502691896258

_NC, _NS, _L = 2, 16, 16
_NW = _NC * _NS
_NBUF = 5
_E = 10

_B_TC = 88000
_TB = 400


@functools.lru_cache(maxsize=None)
def _tp_sc_kernel(B, B_SC, C):
    XW = 4 * C
    OW = 8 * C
    b_per_w = B_SC // _NW
    E = _E
    assert B_SC % _NW == 0 and b_per_w % (_NBUF * E) == 0
    n_supers = b_per_w // E // _NBUF
    G = C // _L

    mesh = plsc.VectorSubcoreMesh(
        core_axis_name="c", subcore_axis_name="s",
        num_cores=_NC, num_subcores=_NS)

    @functools.partial(
        pl.kernel,
        out_type=jax.ShapeDtypeStruct((B * OW,), jnp.float32),
        mesh=mesh,
        scratch_types=(
            [pltpu.VMEM((E * XW,), jnp.float32) for _ in range(_NBUF)]
            + [pltpu.VMEM((E * XW,), jnp.float32) for _ in range(_NBUF)]
            + [pltpu.VMEM((E * OW,), jnp.float32) for _ in range(_NBUF)]
            + [pltpu.SemaphoreType.DMA for _ in range(3 * _NBUF)]
        ),
    )
    def k(x_hbm, y_hbm, o_hbm, *bufs):
        xvs = bufs[0:_NBUF]
        yvs = bufs[_NBUF:2 * _NBUF]
        ovs = bufs[2 * _NBUF:3 * _NBUF]
        sxs = bufs[3 * _NBUF:4 * _NBUF]
        sys_ = bufs[4 * _NBUF:5 * _NBUF]
        sos = bufs[5 * _NBUF:6 * _NBUF]

        wid = lax.axis_index("s") * _NC + lax.axis_index("c")
        base = wid * b_per_w

        def start_in(b, ci):
            e0 = (base + ci * E) * XW
            pltpu.make_async_copy(
                x_hbm.at[pl.ds(e0, E * XW)], xvs[b], sxs[b]).start()
            pltpu.make_async_copy(
                y_hbm.at[pl.ds(e0, E * XW)], yvs[b], sys_[b]).start()

        def wait_in(b):
            pltpu.make_async_copy(
                x_hbm.at[pl.ds(0, E * XW)], xvs[b], sxs[b]).wait()
            pltpu.make_async_copy(
                y_hbm.at[pl.ds(0, E * XW)], yvs[b], sys_[b]).wait()

        def start_out(b, ci):
            e0 = (base + ci * E) * OW
            pltpu.make_async_copy(
                ovs[b], o_hbm.at[pl.ds(e0, E * OW)], sos[b]).start()

        def wait_out(b):
            pltpu.make_async_copy(
                ovs[b], o_hbm.at[pl.ds(0, E * OW)], sos[b]).wait()

        def compute(b):
            xv, yv, ov = xvs[b], yvs[b], ovs[b]

            @plsc.parallel_loop(0, E, step=1, unroll=2)
            def edge_body(e):
                xb = e * XW
                ob = e * OW
                for g in range(G):
                    c0 = g * _L
                    x0 = xv[pl.ds(xb + 0 * C + c0, _L)]
                    x1 = xv[pl.ds(xb + 1 * C + c0, _L)]
                    x2 = xv[pl.ds(xb + 2 * C + c0, _L)]
                    x3 = xv[pl.ds(xb + 3 * C + c0, _L)]
                    y0 = yv[pl.ds(xb + 0 * C + c0, _L)]
                    y1 = yv[pl.ds(xb + 1 * C + c0, _L)]
                    y2 = yv[pl.ds(xb + 2 * C + c0, _L)]
                    y3 = yv[pl.ds(xb + 3 * C + c0, _L)]
                    ov[pl.ds(ob + 0 * C + c0, _L)] = x0 * y0
                    ov[pl.ds(ob + 1 * C + c0, _L)] = x0 * y1
                    ov[pl.ds(ob + 2 * C + c0, _L)] = x0 * y2
                    ov[pl.ds(ob + 3 * C + c0, _L)] = x0 * y3
                    ov[pl.ds(ob + 4 * C + c0, _L)] = x1 * y0
                    ov[pl.ds(ob + 5 * C + c0, _L)] = x2 * y0
                    ov[pl.ds(ob + 6 * C + c0, _L)] = x3 * y0
                    ov[pl.ds(ob + 7 * C + c0, _L)] = (
                        x1 * y1 + x2 * y2 + x3 * y3) * _SQ3

        for b in range(_NBUF):
            start_in(b, b)

        for b in range(_NBUF):
            wait_in(b)
            compute(b)
            start_out(b, b)
            start_in(b, b + _NBUF)

        def super_body(si, carry):
            for b in range(_NBUF):
                ci = si * _NBUF + b
                wait_in(b)
                wait_out(b)
                compute(b)
                start_out(b, ci)
                start_in(b, ci + _NBUF)
            return carry

        lax.fori_loop(1, n_supers - 1, super_body, 0)

        for b in range(_NBUF):
            wait_in(b)
            wait_out(b)
            compute(b)
            start_out(b, (n_supers - 1) * _NBUF + b)
        for b in range(_NBUF):
            wait_out(b)

    return k


def _tc_body(x_ref, y_ref, o_ref):
    x0 = x_ref[:, 0, :]
    x1 = x_ref[:, 1, :]
    x2 = x_ref[:, 2, :]
    x3 = x_ref[:, 3, :]
    y0 = y_ref[:, 0, :]
    y1 = y_ref[:, 1, :]
    y2 = y_ref[:, 2, :]
    y3 = y_ref[:, 3, :]
    o_ref[:, 0, :] = x0 * y0
    o_ref[:, 1, :] = x0 * y1
    o_ref[:, 2, :] = x0 * y2
    o_ref[:, 3, :] = x0 * y3
    o_ref[:, 4, :] = x1 * y0
    o_ref[:, 5, :] = x2 * y0
    o_ref[:, 6, :] = x3 * y0
    o_ref[:, 7, :] = (x1 * y1 + x2 * y2 + x3 * y3) * _SQ3


@functools.lru_cache(maxsize=None)
def _tp_tc_kernel(B, B_SC, B_TC, C):
    assert B_TC % _TB == 0 and B_SC % _TB == 0
    blk0 = B_SC // _TB
    grid = (B_TC // _TB,)
    return pl.pallas_call(
        _tc_body,
        grid=grid,
        in_specs=[
            pl.BlockSpec((_TB, 4, C), lambda i: (blk0 + i, 0, 0)),
            pl.BlockSpec((_TB, 4, C), lambda i: (blk0 + i, 0, 0)),
        ],
        out_specs=pl.BlockSpec((_TB, 8, C), lambda i: (i, 0, 0)),
        out_shape=jax.ShapeDtypeStruct((B_TC, 8, C), jnp.float32),
    )


def kernel(x, y):
    B, _, C = x.shape
    B_TC = _B_TC
    B_SC = B - B_TC
    sc_full = _tp_sc_kernel(B, B_SC, C)(x.reshape(-1), y.reshape(-1))
    tc_part = _tp_tc_kernel(B, B_SC, B_TC, C)(x, y)
    out = lax.dynamic_update_slice(
        sc_full.reshape(B, 8, C), tc_part, (B_SC, 0, 0))
    return out


# speedup vs baseline: 4.6089x; 1.6800x over previous
import functools

import jax
import jax.numpy as jnp
from jax import lax
from jax.experimental import pallas as pl
from jax.experimental.pallas import tpu as pltpu, tpu_sc as plsc

_SQ3 = 0.5773502691896258

_NC, _NS, _L = 2, 16, 16
_NW = _NC * _NS
_NBUF = 5
_E = 10


@functools.lru_cache(maxsize=None)
def _tp_kernel(B, C):
    XW = 4 * C
    OW = 8 * C
    b_per_w = B // _NW
    E = _E
    assert B % _NW == 0 and b_per_w % (_NBUF * E) == 0
    n_chunks = b_per_w // E
    n_supers = n_chunks // _NBUF
    G = C // _L

    mesh = plsc.VectorSubcoreMesh(
        core_axis_name="c", subcore_axis_name="s",
        num_cores=_NC, num_subcores=_NS)

    @functools.partial(
        pl.kernel,
        out_type=jax.ShapeDtypeStruct((B * OW,), jnp.float32),
        mesh=mesh,
        scratch_types=(
            [pltpu.VMEM((E * XW,), jnp.float32) for _ in range(_NBUF)]
            + [pltpu.VMEM((E * XW,), jnp.float32) for _ in range(_NBUF)]
            + [pltpu.VMEM((E * OW,), jnp.float32) for _ in range(_NBUF)]
            + [pltpu.VMEM_SHARED((_NS, E * XW), jnp.float32)
               for _ in range(_NBUF)]
            + [pltpu.SemaphoreType.DMA for _ in range(4 * _NBUF)]
        ),
    )
    def k(x_hbm, y_hbm, o_hbm, *bufs):
        xvs = bufs[0:_NBUF]
        yvs = bufs[_NBUF:2 * _NBUF]
        ovs = bufs[2 * _NBUF:3 * _NBUF]
        ysp = bufs[3 * _NBUF:4 * _NBUF]
        sxs = bufs[4 * _NBUF:5 * _NBUF]
        sy1 = bufs[5 * _NBUF:6 * _NBUF]
        sy2 = bufs[6 * _NBUF:7 * _NBUF]
        sos = bufs[7 * _NBUF:8 * _NBUF]

        sid = lax.axis_index("s")
        wid = sid * _NC + lax.axis_index("c")
        base = wid * b_per_w

        def start_in_x(b, ci):
            e0 = (base + ci * E) * XW
            pltpu.make_async_copy(
                x_hbm.at[pl.ds(e0, E * XW)], xvs[b], sxs[b]).start()

        def wait_in_x(b):
            pltpu.make_async_copy(
                x_hbm.at[pl.ds(0, E * XW)], xvs[b], sxs[b]).wait()

        def start_y1(b, ci):
            e0 = (base + ci * E) * XW
            pltpu.make_async_copy(
                y_hbm.at[pl.ds(e0, E * XW)], ysp[b].at[sid], sy1[b]).start()

        def wait_y1(b):
            pltpu.make_async_copy(
                y_hbm.at[pl.ds(0, E * XW)], ysp[b].at[sid], sy1[b]).wait()

        def start_y2(b):
            pltpu.make_async_copy(ysp[b].at[sid], yvs[b], sy2[b]).start()

        def wait_y2(b):
            pltpu.make_async_copy(ysp[b].at[sid], yvs[b], sy2[b]).wait()

        def start_out(b, ci):
            e0 = (base + ci * E) * OW
            pltpu.make_async_copy(
                ovs[b], o_hbm.at[pl.ds(e0, E * OW)], sos[b]).start()

        def wait_out(b):
            pltpu.make_async_copy(
                ovs[b], o_hbm.at[pl.ds(0, E * OW)], sos[b]).wait()

        def compute(b):
            xv, yv, ov = xvs[b], yvs[b], ovs[b]

            @plsc.parallel_loop(0, E, step=1, unroll=2)
            def edge_body(e):
                xb = e * XW
                ob = e * OW
                for g in range(G):
                    c0 = g * _L
                    x0 = xv[pl.ds(xb + 0 * C + c0, _L)]
                    x1 = xv[pl.ds(xb + 1 * C + c0, _L)]
                    x2 = xv[pl.ds(xb + 2 * C + c0, _L)]
                    x3 = xv[pl.ds(xb + 3 * C + c0, _L)]
                    y0 = yv[pl.ds(xb + 0 * C + c0, _L)]
                    y1 = yv[pl.ds(xb + 1 * C + c0, _L)]
                    y2 = yv[pl.ds(xb + 2 * C + c0, _L)]
                    y3 = yv[pl.ds(xb + 3 * C + c0, _L)]
                    ov[pl.ds(ob + 0 * C + c0, _L)] = x0 * y0
                    ov[pl.ds(ob + 1 * C + c0, _L)] = x0 * y1
                    ov[pl.ds(ob + 2 * C + c0, _L)] = x0 * y2
                    ov[pl.ds(ob + 3 * C + c0, _L)] = x0 * y3
                    ov[pl.ds(ob + 4 * C + c0, _L)] = x1 * y0
                    ov[pl.ds(ob + 5 * C + c0, _L)] = x2 * y0
                    ov[pl.ds(ob + 6 * C + c0, _L)] = x3 * y0
                    ov[pl.ds(ob + 7 * C + c0, _L)] = (
                        x1 * y1 + x2 * y2 + x3 * y3) * _SQ3

        for b in range(_NBUF):
            start_in_x(b, b)
            start_y1(b, b)
        wait_y1(0)
        start_y2(0)

        for b in range(_NBUF):
            wait_in_x(b)
            wait_y2(b)
            compute(b)
            start_out(b, b)
            start_y1(b, b + _NBUF)
            b_next = (b + 1) % _NBUF
            wait_y1(b_next)
            start_y2(b_next)
            start_in_x(b, b + _NBUF)

        def super_body(si, carry):
            for b in range(_NBUF):
                ci = si * _NBUF + b
                wait_in_x(b)
                wait_y2(b)
                wait_out(b)
                compute(b)
                start_out(b, ci)
                start_y1(b, ci + _NBUF)
                b_next = (b + 1) % _NBUF
                wait_y1(b_next)
                start_y2(b_next)
                start_in_x(b, ci + _NBUF)
            return carry

        lax.fori_loop(1, n_supers - 1, super_body, 0)

        for b in range(_NBUF):
            wait_in_x(b)
            wait_y2(b)
            wait_out(b)
            compute(b)
            start_out(b, (n_supers - 1) * _NBUF + b)
            if b + 1 < _NBUF:
                wait_y1(b + 1)
                start_y2(b + 1)
        for b in range(_NBUF):
            wait_out(b)

    return k


def kernel(x, y):
    B, _, C = x.shape
    of = _tp_kernel(B, C)(x.reshape(-1), y.reshape(-1))
    return of.reshape(B, 8, C)


# speedup vs baseline: 4.6864x; 1.0168x over previous
import functools

import jax
import jax.numpy as jnp
from jax import lax
from jax.experimental import pallas as pl
from jax.experimental.pallas import tpu as pltpu, tpu_sc as plsc

_SQ3 = 0.5773502691896258

_NC, _NS, _L = 2, 16, 16
_NW = _NC * _NS
_NBUF = 5
_E = 10


@functools.lru_cache(maxsize=None)
def _tp_kernel(B, C):
    XW = 4 * C
    OW = 8 * C
    b_per_w = B // _NW
    E = _E
    assert B % _NW == 0 and b_per_w % (_NBUF * E) == 0
    n_chunks = b_per_w // E
    n_supers = n_chunks // _NBUF
    G = C // _L

    mesh = plsc.VectorSubcoreMesh(
        core_axis_name="c", subcore_axis_name="s",
        num_cores=_NC, num_subcores=_NS)

    @functools.partial(
        pl.kernel,
        out_type=jax.ShapeDtypeStruct((B * OW,), jnp.float32),
        mesh=mesh,
        scratch_types=(
            [pltpu.VMEM((E * XW,), jnp.float32) for _ in range(_NBUF)]
            + [pltpu.VMEM((E * XW,), jnp.float32) for _ in range(_NBUF)]
            + [pltpu.VMEM((E * OW,), jnp.float32) for _ in range(_NBUF)]
            + [pltpu.SemaphoreType.DMA for _ in range(3 * _NBUF)]
        ),
    )
    def k(x_hbm, y_hbm, o_hbm, *bufs):
        xvs = bufs[0:_NBUF]
        yvs = bufs[_NBUF:2 * _NBUF]
        ovs = bufs[2 * _NBUF:3 * _NBUF]
        sxs = bufs[3 * _NBUF:4 * _NBUF]
        sys_ = bufs[4 * _NBUF:5 * _NBUF]
        sos = bufs[5 * _NBUF:6 * _NBUF]

        wid = lax.axis_index("s") * _NC + lax.axis_index("c")
        base = wid * b_per_w

        def start_in(b, ci):
            e0 = (base + ci * E) * XW
            pltpu.make_async_copy(
                x_hbm.at[pl.ds(e0, E * XW)], xvs[b], sxs[b]).start()
            pltpu.make_async_copy(
                y_hbm.at[pl.ds(e0, E * XW)], yvs[b], sys_[b]).start()

        def wait_in(b):
            pltpu.make_async_copy(
                x_hbm.at[pl.ds(0, E * XW)], xvs[b], sxs[b]).wait()
            pltpu.make_async_copy(
                y_hbm.at[pl.ds(0, E * XW)], yvs[b], sys_[b]).wait()

        def start_out(b, ci):
            e0 = (base + ci * E) * OW
            pltpu.make_async_copy(
                ovs[b], o_hbm.at[pl.ds(e0, E * OW)], sos[b]).start()

        def wait_out(b):
            pltpu.make_async_copy(
                ovs[b], o_hbm.at[pl.ds(0, E * OW)], sos[b]).wait()

        def compute(b):
            xv, yv, ov = xvs[b], yvs[b], ovs[b]

            @plsc.parallel_loop(0, E, step=1, unroll=2)
            def edge_body(e):
                xb = e * XW
                ob = e * OW
                for g in range(G):
                    c0 = g * _L
                    x0 = xv[pl.ds(xb + 0 * C + c0, _L)]
                    x1 = xv[pl.ds(xb + 1 * C + c0, _L)]
                    x2 = xv[pl.ds(xb + 2 * C + c0, _L)]
                    x3 = xv[pl.ds(xb + 3 * C + c0, _L)]
                    y0 = yv[pl.ds(xb + 0 * C + c0, _L)]
                    y1 = yv[pl.ds(xb + 1 * C + c0, _L)]
                    y2 = yv[pl.ds(xb + 2 * C + c0, _L)]
                    y3 = yv[pl.ds(xb + 3 * C + c0, _L)]
                    ov[pl.ds(ob + 0 * C + c0, _L)] = x0 * y0
                    ov[pl.ds(ob + 1 * C + c0, _L)] = x0 * y1
                    ov[pl.ds(ob + 2 * C + c0, _L)] = x0 * y2
                    ov[pl.ds(ob + 3 * C + c0, _L)] = x0 * y3
                    ov[pl.ds(ob + 4 * C + c0, _L)] = x1 * y0
                    ov[pl.ds(ob + 5 * C + c0, _L)] = x2 * y0
                    ov[pl.ds(ob + 6 * C + c0, _L)] = x3 * y0
                    ov[pl.ds(ob + 7 * C + c0, _L)] = (
                        x1 * y1 + x2 * y2 + x3 * y3) * _SQ3

        for b in range(_NBUF):
            start_in(b, b)

        for b in range(_NBUF):
            wait_in(b)
            compute(b)
            start_out(b, b)
            start_in(b, b + _NBUF)

        def super_body(si, carry):
            for b in range(_NBUF):
                ci = si * _NBUF + b
                wait_in(b)
                wait_out(b)
                compute(b)
                start_out(b, ci)
                start_in(b, ci + _NBUF)
            return carry

        lax.fori_loop(1, n_supers - 1, super_body, 0)

        for b in range(_NBUF):
            wait_in(b)
            wait_out(b)
            compute(b)
            start_out(b, (n_supers - 1) * _NBUF + b)
        for b in range(_NBUF):
            wait_out(b)

    return k


def kernel(x, y):
    B, _, C = x.shape
    of = _tp_kernel(B, C)(x.reshape(-1), y.reshape(-1))
    return of.reshape(B, 8, C)


# speedup vs baseline: 4.6968x; 1.0022x over previous
import functools

import jax
import jax.numpy as jnp
from jax import lax
from jax.experimental import pallas as pl
from jax.experimental.pallas import tpu as pltpu, tpu_sc as plsc

_SQ3 = 0.5773502691896258

_NC, _NS, _L = 2, 16, 16
_NW = _NC * _NS
_NBUF = 5
_E = 10


@functools.lru_cache(maxsize=None)
def _tp_kernel(B, C):
    XW = 4 * C
    OW = 8 * C
    b_per_w = B // _NW
    E = _E
    assert B % _NW == 0 and b_per_w % (_NBUF * E) == 0
    n_chunks = b_per_w // E
    n_supers = n_chunks // _NBUF
    G = C // _L

    mesh = plsc.VectorSubcoreMesh(
        core_axis_name="c", subcore_axis_name="s",
        num_cores=_NC, num_subcores=_NS)

    @functools.partial(
        pl.kernel,
        out_type=jax.ShapeDtypeStruct((B * OW,), jnp.float32),
        mesh=mesh,
        scratch_types=(
            [pltpu.VMEM((E * XW,), jnp.float32) for _ in range(_NBUF)]
            + [pltpu.VMEM((E * XW,), jnp.float32) for _ in range(_NBUF)]
            + [pltpu.VMEM((E * OW,), jnp.float32) for _ in range(_NBUF)]
            + [pltpu.SemaphoreType.DMA for _ in range(3 * _NBUF)]
        ),
    )
    def k(x_hbm, y_hbm, o_hbm, *bufs):
        xvs = bufs[0:_NBUF]
        yvs = bufs[_NBUF:2 * _NBUF]
        ovs = bufs[2 * _NBUF:3 * _NBUF]
        sxs = bufs[3 * _NBUF:4 * _NBUF]
        sys_ = bufs[4 * _NBUF:5 * _NBUF]
        sos = bufs[5 * _NBUF:6 * _NBUF]

        wid = lax.axis_index("s") * _NC + lax.axis_index("c")
        base = wid * b_per_w

        def start_in(b, ci):
            e0 = (base + ci * E) * XW
            pltpu.make_async_copy(
                x_hbm.at[pl.ds(e0, E * XW)], xvs[b], sxs[b]).start()
            pltpu.make_async_copy(
                y_hbm.at[pl.ds(e0, E * XW)], yvs[b], sys_[b]).start()

        def wait_in(b):
            pltpu.make_async_copy(
                x_hbm.at[pl.ds(0, E * XW)], xvs[b], sxs[b]).wait()
            pltpu.make_async_copy(
                y_hbm.at[pl.ds(0, E * XW)], yvs[b], sys_[b]).wait()

        def start_out(b, ci):
            e0 = (base + ci * E) * OW
            pltpu.make_async_copy(
                ovs[b], o_hbm.at[pl.ds(e0, E * OW)], sos[b]).start()

        def wait_out(b):
            pltpu.make_async_copy(
                ovs[b], o_hbm.at[pl.ds(0, E * OW)], sos[b]).wait()

        def compute(b):
            xv, yv, ov = xvs[b], yvs[b], ovs[b]

            def edge_body(e, c2):
                xb = e * XW
                ob = e * OW
                for g in range(G):
                    c0 = g * _L
                    x0 = xv[pl.ds(xb + 0 * C + c0, _L)]
                    x1 = xv[pl.ds(xb + 1 * C + c0, _L)]
                    x2 = xv[pl.ds(xb + 2 * C + c0, _L)]
                    x3 = xv[pl.ds(xb + 3 * C + c0, _L)]
                    y0 = yv[pl.ds(xb + 0 * C + c0, _L)]
                    y1 = yv[pl.ds(xb + 1 * C + c0, _L)]
                    y2 = yv[pl.ds(xb + 2 * C + c0, _L)]
                    y3 = yv[pl.ds(xb + 3 * C + c0, _L)]
                    ov[pl.ds(ob + 0 * C + c0, _L)] = x0 * y0
                    ov[pl.ds(ob + 1 * C + c0, _L)] = x0 * y1
                    ov[pl.ds(ob + 2 * C + c0, _L)] = x0 * y2
                    ov[pl.ds(ob + 3 * C + c0, _L)] = x0 * y3
                    ov[pl.ds(ob + 4 * C + c0, _L)] = x1 * y0
                    ov[pl.ds(ob + 5 * C + c0, _L)] = x2 * y0
                    ov[pl.ds(ob + 6 * C + c0, _L)] = x3 * y0
                    ov[pl.ds(ob + 7 * C + c0, _L)] = (
                        x1 * y1 + x2 * y2 + x3 * y3) * _SQ3
                return c2

            lax.fori_loop(0, E, edge_body, 0)

        for b in range(_NBUF):
            start_in(b, b)

        for b in range(_NBUF):
            wait_in(b)
            compute(b)
            start_out(b, b)
            start_in(b, b + _NBUF)

        def super_body(si, carry):
            for b in range(_NBUF):
                ci = si * _NBUF + b
                wait_in(b)
                wait_out(b)
                compute(b)
                start_out(b, ci)
                start_in(b, ci + _NBUF)
            return carry

        lax.fori_loop(1, n_supers - 1, super_body, 0)

        for b in range(_NBUF):
            wait_in(b)
            wait_out(b)
            compute(b)
            start_out(b, (n_supers - 1) * _NBUF + b)
        for b in range(_NBUF):
            wait_out(b)

    return k


def kernel(x, y):
    B, _, C = x.shape
    of = _tp_kernel(B, C)(x.reshape(-1), y.reshape(-1))
    return of.reshape(B, 8, C)
